# baseline (speedup 1.0000x reference)
import jax
import jax.numpy as jnp
from jax.experimental import pallas as pl
from jax.experimental.pallas import tpu as pltpu

_SLAB = 64
_NSEM = 8


def _bcast_add_kernel(qpw_ref, q_ref, out_ref, rep_ref, sems):
    s = q_ref[0] + qpw_ref[...]
    rep_ref[...] = jnp.broadcast_to(s[:, None, :], rep_ref.shape)
    bs = out_ref.shape[1]
    n = bs // _SLAB
    copies = [
        pltpu.make_async_copy(
            rep_ref,
            out_ref.at[:, pl.ds(i * _SLAB, _SLAB), :],
            sems.at[i % _NSEM],
        )
        for i in range(n)
    ]
    for c in copies:
        c.start()
    for c in copies:
        c.wait()


def kernel(x, query_pos_weight, queries):
    bs = x.shape[0]
    n_query, embed_dim = query_pos_weight.shape
    out = pl.pallas_call(
        _bcast_add_kernel,
        in_specs=[
            pl.BlockSpec(memory_space=pltpu.VMEM),
            pl.BlockSpec(memory_space=pltpu.VMEM),
        ],
        out_specs=pl.BlockSpec(memory_space=pl.ANY),
        out_shape=jax.ShapeDtypeStruct((n_query, bs, embed_dim), queries.dtype),
        scratch_shapes=[
            pltpu.VMEM((n_query, _SLAB, embed_dim), queries.dtype),
            pltpu.SemaphoreType.DMA((_NSEM,)),
        ],
    )(query_pos_weight, queries)
    return jnp.swapaxes(out, 0, 1)


# speedup vs baseline: 1.0076x; 1.0076x over previous
import jax
import jax.numpy as jnp
from jax.experimental import pallas as pl
from jax.experimental.pallas import tpu as pltpu

_SLAB = 64
_NSEM = 8


def _bcast_add_kernel(qpw_ref, q_ref, out_ref, rep_ref, sems):
    s = q_ref[0] + qpw_ref[...]
    rep_ref[...] = jnp.broadcast_to(s[:, None, :], rep_ref.shape)
    bs = out_ref.shape[1]
    n = bs // _SLAB
    copies = [
        pltpu.make_async_copy(
            rep_ref,
            out_ref.at[:, pl.ds(i * _SLAB, _SLAB), :],
            sems.at[i % _NSEM],
        )
        for i in range(n)
    ]
    w = 3
    for i, c in enumerate(copies):
        if i >= w:
            copies[i - w].wait()
        c.start()
    for c in copies[-w:]:
        c.wait()


def kernel(x, query_pos_weight, queries):
    bs = x.shape[0]
    n_query, embed_dim = query_pos_weight.shape
    out = pl.pallas_call(
        _bcast_add_kernel,
        in_specs=[
            pl.BlockSpec(memory_space=pltpu.VMEM),
            pl.BlockSpec(memory_space=pltpu.VMEM),
        ],
        out_specs=pl.BlockSpec(memory_space=pl.ANY),
        out_shape=jax.ShapeDtypeStruct((n_query, bs, embed_dim), queries.dtype),
        scratch_shapes=[
            pltpu.VMEM((n_query, _SLAB, embed_dim), queries.dtype),
            pltpu.SemaphoreType.DMA((_NSEM,)),
        ],
    )(query_pos_weight, queries)
    return jnp.swapaxes(out, 0, 1)
